# baseline (speedup 1.0000x reference)
import functools

import jax
import jax.numpy as jnp
from jax import lax
from jax.experimental import pallas as pl
from jax.experimental.pallas import tpu as pltpu
from jax.experimental.pallas import tpu_sc as plsc

B = 4096
S = 50
D = 128
NUM_CORES = 2
NUM_SUBCORES = 16
NW = NUM_CORES * NUM_SUBCORES
BPW = B // NW
CHUNK = 128
HPS = BPW // CHUNK
NCHUNK = S * HPS
NBUF = 5
LOOKAHEAD = 2
NBLK = NCHUNK // NBUF


@functools.partial(
    pl.kernel,
    out_type=jax.ShapeDtypeStruct((S, B, D), jnp.float32),
    mesh=plsc.VectorSubcoreMesh(core_axis_name="c", subcore_axis_name="s"),
    scratch_types=[
        pltpu.VMEM((S, BPW), jnp.int32),
        pltpu.VMEM((NBUF, CHUNK, D), jnp.float32),
    ]
    + [pltpu.SemaphoreType.DMA] * (2 * NBUF),
)
def _embed_gather(xtw_hbm, table_hbm, out_hbm, idx_v, rows_v, *sems):
    gsems = sems[:NBUF]
    osems = sems[NBUF:]
    wid = lax.axis_index("s") * NUM_CORES + lax.axis_index("c")
    base = pl.multiple_of(wid * BPW, BPW)
    pltpu.sync_copy(xtw_hbm.at[wid], idx_v)

    def offs(j):
        s = j // HPS
        h = (j % HPS) * CHUNK
        if not isinstance(j, int):
            h = pl.multiple_of(h, CHUNK)
        return s, h

    def g_copy(j, slot):
        s, h = offs(j)
        return pltpu.make_async_copy(
            table_hbm.at[idx_v.at[s, pl.ds(h, CHUNK)]],
            rows_v.at[slot],
            gsems[slot],
        )

    def w_copy(j, slot):
        s, h = offs(j)
        off = h + base if isinstance(j, int) else pl.multiple_of(h + base, CHUNK)
        return pltpu.make_async_copy(
            rows_v.at[slot],
            out_hbm.at[s, pl.ds(off, CHUNK)],
            osems[slot],
        )

    for j in range(LOOKAHEAD):
        g_copy(j, j).start()

    for k in range(NBUF):
        j = k
        g_copy(j, k).wait()
        w_copy(j, k).start()
        jn = j + LOOKAHEAD
        if jn >= NBUF:
            w_copy(jn - NBUF, jn % NBUF).wait()
        g_copy(jn, jn % NBUF).start()

    def body(i, carry):
        for k in range(NBUF):
            j = i * NBUF + k
            slot_n = (k + LOOKAHEAD) % NBUF
            g_copy(j, k).wait()
            w_copy(j, k).start()
            jn = j + LOOKAHEAD
            w_copy(jn - NBUF, slot_n).wait()
            g_copy(jn, slot_n).start()
        return carry

    lax.fori_loop(1, NBLK - 1, body, 0)

    for k in range(NBUF):
        j = (NBLK - 1) * NBUF + k
        g_copy(j, k).wait()
        w_copy(j, k).start()
        jn = j + LOOKAHEAD
        if jn < NCHUNK:
            w_copy(jn - NBUF, jn % NBUF).wait()
            g_copy(jn, jn % NBUF).start()
    for k in range(NBUF):
        j = (NBLK - 1) * NBUF + k
        w_copy(j, k).wait()


def kernel(x, table):
    xtw = x.astype(jnp.int32).T.reshape(S, NW, BPW).transpose(1, 0, 2)
    out_sbd = _embed_gather(xtw, table)
    return out_sbd.transpose(1, 0, 2)


# speedup vs baseline: 1.0153x; 1.0153x over previous
import functools

import jax
import jax.numpy as jnp
from jax import lax
from jax.experimental import pallas as pl
from jax.experimental.pallas import tpu as pltpu
from jax.experimental.pallas import tpu_sc as plsc

B = 4096
S = 50
D = 128
NUM_CORES = 2
NUM_SUBCORES = 16
NW = NUM_CORES * NUM_SUBCORES
BPW = B // NW
CHUNK = 64
HPS = BPW // CHUNK
NCHUNK = S * HPS
NBUF = 10
LOOKAHEAD = 6
NBLK = NCHUNK // NBUF


@functools.partial(
    pl.kernel,
    out_type=jax.ShapeDtypeStruct((S, B, D), jnp.float32),
    mesh=plsc.VectorSubcoreMesh(core_axis_name="c", subcore_axis_name="s"),
    scratch_types=[
        pltpu.VMEM((S, BPW), jnp.int32),
        pltpu.VMEM((NBUF, CHUNK, D), jnp.float32),
    ]
    + [pltpu.SemaphoreType.DMA] * (2 * NBUF),
)
def _embed_gather(xtw_hbm, table_hbm, out_hbm, idx_v, rows_v, *sems):
    gsems = sems[:NBUF]
    osems = sems[NBUF:]
    wid = lax.axis_index("s") * NUM_CORES + lax.axis_index("c")
    base = pl.multiple_of(wid * BPW, BPW)
    pltpu.sync_copy(xtw_hbm.at[wid], idx_v)

    def offs(j):
        s = j // HPS
        h = (j % HPS) * CHUNK
        if not isinstance(j, int):
            h = pl.multiple_of(h, CHUNK)
        return s, h

    def g_copy(j, slot):
        s, h = offs(j)
        return pltpu.make_async_copy(
            table_hbm.at[idx_v.at[s, pl.ds(h, CHUNK)]],
            rows_v.at[slot],
            gsems[slot],
        )

    def w_copy(j, slot):
        s, h = offs(j)
        off = h + base if isinstance(j, int) else pl.multiple_of(h + base, CHUNK)
        return pltpu.make_async_copy(
            rows_v.at[slot],
            out_hbm.at[s, pl.ds(off, CHUNK)],
            osems[slot],
        )

    for j in range(LOOKAHEAD):
        g_copy(j, j).start()

    for k in range(NBUF):
        j = k
        g_copy(j, k).wait()
        w_copy(j, k).start()
        jn = j + LOOKAHEAD
        if jn >= NBUF:
            w_copy(jn - NBUF, jn % NBUF).wait()
        g_copy(jn, jn % NBUF).start()

    def body(i, carry):
        for k in range(NBUF):
            j = i * NBUF + k
            slot_n = (k + LOOKAHEAD) % NBUF
            g_copy(j, k).wait()
            w_copy(j, k).start()
            jn = j + LOOKAHEAD
            w_copy(jn - NBUF, slot_n).wait()
            g_copy(jn, slot_n).start()
        return carry

    lax.fori_loop(1, NBLK - 1, body, 0)

    for k in range(NBUF):
        j = (NBLK - 1) * NBUF + k
        g_copy(j, k).wait()
        w_copy(j, k).start()
        jn = j + LOOKAHEAD
        if jn < NCHUNK:
            w_copy(jn - NBUF, jn % NBUF).wait()
            g_copy(jn, jn % NBUF).start()
    for k in range(NBUF):
        j = (NBLK - 1) * NBUF + k
        w_copy(j, k).wait()


def kernel(x, table):
    xtw = x.astype(jnp.int32).T.reshape(S, NW, BPW).transpose(1, 0, 2)
    out_sbd = _embed_gather(xtw, table)
    return out_sbd.transpose(1, 0, 2)


# speedup vs baseline: 1.0165x; 1.0011x over previous
import functools

import jax
import jax.numpy as jnp
from jax import lax
from jax.experimental import pallas as pl
from jax.experimental.pallas import tpu as pltpu
from jax.experimental.pallas import tpu_sc as plsc

B = 4096
S = 50
D = 128
NUM_CORES = 2
NUM_SUBCORES = 16
NW = NUM_CORES * NUM_SUBCORES
BPW = B // NW
CHUNK = 64
HPS = BPW // CHUNK
NCHUNK = S * HPS
NBUF = 10
LOOKAHEAD = 7
NBLK = NCHUNK // NBUF


@functools.partial(
    pl.kernel,
    out_type=jax.ShapeDtypeStruct((S, B, D), jnp.float32),
    mesh=plsc.VectorSubcoreMesh(core_axis_name="c", subcore_axis_name="s"),
    scratch_types=[
        pltpu.VMEM((S, BPW), jnp.int32),
        pltpu.VMEM((NBUF, CHUNK, D), jnp.float32),
    ]
    + [pltpu.SemaphoreType.DMA] * (2 * NBUF),
)
def _embed_gather(xtw_hbm, table_hbm, out_hbm, idx_v, rows_v, *sems):
    gsems = sems[:NBUF]
    osems = sems[NBUF:]
    wid = lax.axis_index("s") * NUM_CORES + lax.axis_index("c")
    base = pl.multiple_of(wid * BPW, BPW)
    pltpu.sync_copy(xtw_hbm.at[wid], idx_v)

    def offs(j):
        s = j // HPS
        h = (j % HPS) * CHUNK
        if not isinstance(j, int):
            h = pl.multiple_of(h, CHUNK)
        return s, h

    def g_copy(j, slot):
        s, h = offs(j)
        return pltpu.make_async_copy(
            table_hbm.at[idx_v.at[s, pl.ds(h, CHUNK)]],
            rows_v.at[slot],
            gsems[slot],
        )

    def w_copy(j, slot):
        s, h = offs(j)
        off = h + base if isinstance(j, int) else pl.multiple_of(h + base, CHUNK)
        return pltpu.make_async_copy(
            rows_v.at[slot],
            out_hbm.at[s, pl.ds(off, CHUNK)],
            osems[slot],
        )

    for j in range(LOOKAHEAD):
        g_copy(j, j).start()

    for k in range(NBUF):
        j = k
        g_copy(j, k).wait()
        w_copy(j, k).start()
        jn = j + LOOKAHEAD
        if jn >= NBUF:
            w_copy(jn - NBUF, jn % NBUF).wait()
        g_copy(jn, jn % NBUF).start()

    def body(i, carry):
        for k in range(NBUF):
            j = i * NBUF + k
            slot_n = (k + LOOKAHEAD) % NBUF
            g_copy(j, k).wait()
            w_copy(j, k).start()
            jn = j + LOOKAHEAD
            w_copy(jn - NBUF, slot_n).wait()
            g_copy(jn, slot_n).start()
        return carry

    lax.fori_loop(1, NBLK - 1, body, 0)

    for k in range(NBUF):
        j = (NBLK - 1) * NBUF + k
        g_copy(j, k).wait()
        w_copy(j, k).start()
        jn = j + LOOKAHEAD
        if jn < NCHUNK:
            w_copy(jn - NBUF, jn % NBUF).wait()
            g_copy(jn, jn % NBUF).start()
    for k in range(NBUF):
        j = (NBLK - 1) * NBUF + k
        w_copy(j, k).wait()


def kernel(x, table):
    xtw = x.astype(jnp.int32).T.reshape(S, NW, BPW).transpose(1, 0, 2)
    out_sbd = _embed_gather(xtw, table)
    return out_sbd.transpose(1, 0, 2)
